# Initial kernel scaffold; baseline (speedup 1.0000x reference)
import jax
import jax.numpy as jnp
from jax.experimental import pallas as pl


def kernel(xyz, params):
    raise NotImplementedError("write your pallas kernel here")


# speedup vs baseline: 4.0973x; 4.0973x over previous
import functools
import numpy as np
import jax
import jax.numpy as jnp
from jax.experimental import pallas as pl

_SPECS = [
    (1024, [0.025, 0.05], [16, 32], [[3, 32, 32], [3, 32, 32]]),
    (512, [0.05, 0.1], [16, 32], [[64, 64, 128], [64, 64, 128]]),
    (256, [0.1, 0.2], [16, 32], [[256, 196, 256], [256, 196, 256]]),
]


def _fps_body(npoint, xyz_ref, out_ref):
    xyz = xyz_ref[...]
    b, _, n = xyz.shape
    iota_n = jax.lax.broadcasted_iota(jnp.int32, (b, n), 1)
    iota_p = jax.lax.broadcasted_iota(jnp.int32, (b, 3, npoint), 2)

    def step(i, carry):
        distance, far, acc = carry
        sel = (iota_n == far).astype(xyz.dtype)
        c = jnp.sum(xyz * sel[:, None, :], axis=2)
        d = xyz - c[:, :, None]
        dist = jnp.sum(d * d, axis=1)
        distance = jnp.minimum(distance, dist)
        m = jnp.max(distance, axis=1, keepdims=True)
        far2 = jnp.min(jnp.where(distance == m, iota_n, n), axis=1,
                       keepdims=True)
        acc = jnp.where(iota_p == i, c[:, :, None], acc)
        return distance, far2, acc

    init = (jnp.full((b, n), 1e10, jnp.float32),
            jnp.zeros((b, 1), jnp.int32),
            jnp.zeros((b, 3, npoint), jnp.float32))
    _, _, acc = jax.lax.fori_loop(0, npoint, step, init)
    out_ref[...] = acc


def _fps(xyz, npoint):
    b, _, n = xyz.shape
    return pl.pallas_call(
        functools.partial(_fps_body, npoint),
        out_shape=jax.ShapeDtypeStruct((b, 3, npoint), jnp.float32),
    )(xyz)


def _cumsum_lanes(x):
    n = x.shape[-1]
    s = 1
    while s < n:
        shifted = jnp.concatenate(
            [jnp.zeros(x.shape[:-1] + (s,), x.dtype), x[..., :-s]], axis=-1)
        x = x + shifted
        s *= 2
    return x


def _group_body(k, rsq, nfeat, xyz_ref, data_ref, cent_ref, w_ref, b_ref,
                y_ref, st_ref):
    ts = cent_ref.shape[2]
    xyz = xyz_ref[0]
    data = data_ref[0]
    cent = cent_ref[0]
    ct = cent.T

    sq = -2.0 * jnp.dot(ct.astype(jnp.bfloat16), xyz.astype(jnp.bfloat16),
                        preferred_element_type=jnp.float32)
    sq = sq + jnp.sum(ct * ct, axis=1, keepdims=True)
    sq = sq + jnp.sum(xyz * xyz, axis=0, keepdims=True)

    mask = sq <= rsq
    rank = _cumsum_lanes(mask.astype(jnp.int32))
    cnt = rank[:, -1:]
    sub = jnp.concatenate(
        [jnp.zeros((ts, nfeat), jnp.float32), ct], axis=1)

    w = w_ref[...]
    bb = b_ref[...]
    o1 = w.shape[1]

    first = (pl.program_id(0) == 0) & (pl.program_id(1) == 0)

    @pl.when(first)
    def _():
        st_ref[...] = jnp.zeros_like(st_ref)

    n = xyz.shape[1]
    lastcol = jnp.logical_and(
        cnt == 0,
        jax.lax.broadcasted_iota(jnp.int32, (ts, n), 1) == n - 1)

    s1 = jnp.zeros((1, o1), jnp.float32)
    s2 = jnp.zeros((1, o1), jnp.float32)
    for j in range(k):
        tr = jnp.where(cnt >= (j + 1), j + 1, 1)
        oh = jnp.logical_or(
            jnp.logical_and(mask, rank == tr), lastcol).astype(jnp.float32)
        g = jnp.dot(oh, data, preferred_element_type=jnp.float32,
                precision=jax.lax.Precision.HIGHEST) - sub
        y = jnp.dot(g, w, preferred_element_type=jnp.float32) + bb
        y_ref[0, 0, j * ts:(j + 1) * ts, :] = y
        s1 = s1 + jnp.sum(y, axis=0, keepdims=True)
        s2 = s2 + jnp.sum(y * y, axis=0, keepdims=True)
    st_ref[0:1, :] += s1
    st_ref[1:2, :] += s2


def _group_conv1(xyz, data_t, centers, w1t, b1, k, rsq, ts):
    b, _, n = xyz.shape
    s = centers.shape[2]
    cdim = data_t.shape[2]
    o1 = w1t.shape[1]
    nst = s // ts
    y, st = pl.pallas_call(
        functools.partial(_group_body, k, rsq, cdim - 3),
        grid=(b, nst),
        in_specs=[
            pl.BlockSpec((1, 3, n), lambda i, j: (i, 0, 0)),
            pl.BlockSpec((1, n, cdim), lambda i, j: (i, 0, 0)),
            pl.BlockSpec((1, 3, ts), lambda i, j: (i, 0, j)),
            pl.BlockSpec((cdim, o1), lambda i, j: (0, 0)),
            pl.BlockSpec((1, o1), lambda i, j: (0, 0)),
        ],
        out_specs=[
            pl.BlockSpec((1, 1, k * ts, o1), lambda i, j: (i, j, 0, 0)),
            pl.BlockSpec((2, o1), lambda i, j: (0, 0)),
        ],
        out_shape=[
            jax.ShapeDtypeStruct((b, nst, k * ts, o1), jnp.float32),
            jax.ShapeDtypeStruct((2, o1), jnp.float32),
        ],
    )(xyz, data_t, centers, w1t, b1)
    return y.reshape(b * s * k, o1), st


def _conv_body(x_ref, a_ref, c_ref, w_ref, b_ref, y_ref, st_ref):
    x = x_ref[...]
    z = jnp.maximum(x * a_ref[...] + c_ref[...], 0.0)
    y = jnp.dot(z, w_ref[...], preferred_element_type=jnp.float32) + b_ref[...]
    y_ref[...] = y

    @pl.when(pl.program_id(0) == 0)
    def _():
        st_ref[...] = jnp.zeros_like(st_ref)

    st_ref[0:1, :] += jnp.sum(y, axis=0, keepdims=True)
    st_ref[1:2, :] += jnp.sum(y * y, axis=0, keepdims=True)


def _conv_next(x, a, c, wt, bb, tr):
    r, cin = x.shape
    o = wt.shape[1]
    y, st = pl.pallas_call(
        _conv_body,
        grid=(r // tr,),
        in_specs=[
            pl.BlockSpec((tr, cin), lambda i: (i, 0)),
            pl.BlockSpec((1, cin), lambda i: (0, 0)),
            pl.BlockSpec((1, cin), lambda i: (0, 0)),
            pl.BlockSpec((cin, o), lambda i: (0, 0)),
            pl.BlockSpec((1, o), lambda i: (0, 0)),
        ],
        out_specs=[
            pl.BlockSpec((tr, o), lambda i: (i, 0)),
            pl.BlockSpec((2, o), lambda i: (0, 0)),
        ],
        out_shape=[
            jax.ShapeDtypeStruct((r, o), jnp.float32),
            jax.ShapeDtypeStruct((2, o), jnp.float32),
        ],
    )(x, a, c, wt, bb)
    return y, st


def _pool_body(k, ts, x_ref, a_ref, c_ref, o_ref):
    x = x_ref[0]
    z = jnp.maximum(x * a_ref[...] + c_ref[...], 0.0)
    m = z[0:ts, :]
    for j in range(1, k):
        m = jnp.maximum(m, z[j * ts:(j + 1) * ts, :])
    o_ref[0] = m


def _pool(y, a, c, b, s, k, ts):
    o = y.shape[-1]
    nst = (b * s) // ts
    y4 = y.reshape(nst, k * ts, o)
    out = pl.pallas_call(
        functools.partial(_pool_body, k, ts),
        grid=(nst,),
        in_specs=[
            pl.BlockSpec((1, k * ts, o), lambda i: (i, 0, 0)),
            pl.BlockSpec((1, o), lambda i: (0, 0)),
            pl.BlockSpec((1, o), lambda i: (0, 0)),
        ],
        out_specs=pl.BlockSpec((1, ts, o), lambda i: (i, 0, 0)),
        out_shape=jax.ShapeDtypeStruct((nst, ts, o), jnp.float32),
    )(y4, a, c)
    return out.reshape(b, s, o)


def _bn_affine(st, m, gamma, beta):
    mean = st[0] / m
    var = st[1] / m - mean * mean
    a = gamma / jnp.sqrt(var + 1e-5)
    c = beta - mean * a
    return a[None, :], c[None, :]


def _sa_layer(xyz, feats_t, centers, spec, branch_params, ts):
    b = xyz.shape[0]
    s = centers.shape[2]
    _, radius_list, nsample_list, _ = spec
    xyz_t = jnp.transpose(xyz, (0, 2, 1))
    if feats_t is None:
        data_t = jnp.concatenate([xyz_t, xyz_t], axis=2)
    else:
        data_t = jnp.concatenate([feats_t, xyz_t], axis=2)
    outs = []
    for i, radius in enumerate(radius_list):
        k = nsample_list[i]
        rsq = np.float32(radius ** 2)
        layers = branch_params[i]
        (w1, b1, g1, be1) = layers[0]
        y, st = _group_conv1(xyz, data_t, centers, jnp.transpose(w1),
                             b1[None, :], k, rsq, ts)
        m = np.float32(y.shape[0])
        a, c = _bn_affine(st, m, g1, be1)
        for (w, bb, gm, bt) in layers[1:]:
            y, st = _conv_next(y, a, c, jnp.transpose(w), bb[None, :],
                               min(2048, y.shape[0]))
            a, c = _bn_affine(st, m, gm, bt)
        pooled = _pool(y, a, c, b, s, k, ts)
        outs.append(jnp.transpose(pooled, (0, 2, 1)))
    return jnp.concatenate(outs, axis=1)


def kernel(xyz, params):
    l1_xyz = _fps(xyz, _SPECS[0][0])
    l1_pts = _sa_layer(xyz, None, l1_xyz, _SPECS[0], params[0], 128)

    l2_xyz = _fps(l1_xyz, _SPECS[1][0])
    l1_pts_t = jnp.transpose(l1_pts, (0, 2, 1))
    l2_pts = _sa_layer(l1_xyz, l1_pts_t, l2_xyz, _SPECS[1], params[1], 128)

    l3_xyz = _fps(l2_xyz, _SPECS[2][0])
    l2_pts_t = jnp.transpose(l2_pts, (0, 2, 1))
    l3_pts = _sa_layer(l2_xyz, l2_pts_t, l3_xyz, _SPECS[2], params[2], 128)

    return (l1_xyz, l2_xyz, l3_xyz, l3_pts, l2_pts, l1_pts)


# speedup vs baseline: 4.5144x; 1.1018x over previous
import functools
import numpy as np
import jax
import jax.numpy as jnp
from jax.experimental import pallas as pl

_SPECS = [
    (1024, [0.025, 0.05], [16, 32], [[3, 32, 32], [3, 32, 32]]),
    (512, [0.05, 0.1], [16, 32], [[64, 64, 128], [64, 64, 128]]),
    (256, [0.1, 0.2], [16, 32], [[256, 196, 256], [256, 196, 256]]),
]


def _fps_body(npoint, xyz_ref, out_ref):
    xyz = xyz_ref[...]
    b, _, n = xyz.shape
    iota_n = jax.lax.broadcasted_iota(jnp.int32, (b, n), 1)
    iota_p = jax.lax.broadcasted_iota(jnp.int32, (b, 3, npoint), 2)

    def step(i, carry):
        distance, far, acc = carry
        sel = (iota_n == far).astype(xyz.dtype)
        c = jnp.sum(xyz * sel[:, None, :], axis=2)
        d = xyz - c[:, :, None]
        dist = jnp.sum(d * d, axis=1)
        distance = jnp.minimum(distance, dist)
        m = jnp.max(distance, axis=1, keepdims=True)
        far2 = jnp.min(jnp.where(distance == m, iota_n, n), axis=1,
                       keepdims=True)
        acc = jnp.where(iota_p == i, c[:, :, None], acc)
        return distance, far2, acc

    init = (jnp.full((b, n), 1e10, jnp.float32),
            jnp.zeros((b, 1), jnp.int32),
            jnp.zeros((b, 3, npoint), jnp.float32))
    _, _, acc = jax.lax.fori_loop(0, npoint, step, init)
    out_ref[...] = acc


def _fps(xyz, npoint):
    b, _, n = xyz.shape
    return pl.pallas_call(
        functools.partial(_fps_body, npoint),
        out_shape=jax.ShapeDtypeStruct((b, 3, npoint), jnp.float32),
    )(xyz)


def _cumsum_lanes(x):
    n = x.shape[-1]
    s = 1
    while s < n:
        shifted = jnp.concatenate(
            [jnp.zeros(x.shape[:-1] + (s,), x.dtype), x[..., :-s]], axis=-1)
        x = x + shifted
        s *= 2
    return x


def _group_body(k, rsq, nfeat, xyz_ref, data_ref, cent_ref, w_ref, b_ref,
                y_ref, st_ref):
    ts = cent_ref.shape[2]
    xyz = xyz_ref[0]
    data = data_ref[0]
    cent = cent_ref[0]
    ct = cent.T

    sq = -2.0 * jnp.dot(ct.astype(jnp.bfloat16), xyz.astype(jnp.bfloat16),
                        preferred_element_type=jnp.float32)
    sq = sq + jnp.sum(ct * ct, axis=1, keepdims=True)
    sq = sq + jnp.sum(xyz * xyz, axis=0, keepdims=True)

    mask = sq <= rsq
    rank = _cumsum_lanes(mask.astype(jnp.int32))
    cnt = rank[:, -1:]
    sub = jnp.concatenate(
        [jnp.zeros((ts, nfeat), jnp.float32), ct], axis=1)

    w = w_ref[...]
    bb = b_ref[...]
    o1 = w.shape[1]

    first = (pl.program_id(0) == 0) & (pl.program_id(1) == 0)

    @pl.when(first)
    def _():
        st_ref[...] = jnp.zeros_like(st_ref)

    n = xyz.shape[1]
    lane = jax.lax.broadcasted_iota(jnp.int32, (ts, n), 1)
    rank2 = jnp.where(mask, rank, 0)
    rank2 = jnp.where(jnp.logical_and(cnt == 0, lane == n - 1),
                     1, rank2)

    s1 = jnp.zeros((1, o1), jnp.float32)
    s2 = jnp.zeros((1, o1), jnp.float32)
    for j in range(k):
        tr = jnp.where(cnt >= (j + 1), j + 1, 1)
        oh = (rank2 == tr).astype(jnp.float32)
        g = jnp.dot(oh, data, preferred_element_type=jnp.float32,
                precision=jax.lax.Precision.HIGHEST) - sub
        y = jnp.dot(g, w, preferred_element_type=jnp.float32) + bb
        y_ref[0, 0, j * ts:(j + 1) * ts, :] = y
        s1 = s1 + jnp.sum(y, axis=0, keepdims=True)
        s2 = s2 + jnp.sum(y * y, axis=0, keepdims=True)
    st_ref[0:1, :] += s1
    st_ref[1:2, :] += s2


def _group_conv1(xyz, data_t, centers, w1t, b1, k, rsq, ts):
    b, _, n = xyz.shape
    s = centers.shape[2]
    cdim = data_t.shape[2]
    o1 = w1t.shape[1]
    nst = s // ts
    y, st = pl.pallas_call(
        functools.partial(_group_body, k, rsq, cdim - 3),
        grid=(b, nst),
        in_specs=[
            pl.BlockSpec((1, 3, n), lambda i, j: (i, 0, 0)),
            pl.BlockSpec((1, n, cdim), lambda i, j: (i, 0, 0)),
            pl.BlockSpec((1, 3, ts), lambda i, j: (i, 0, j)),
            pl.BlockSpec((cdim, o1), lambda i, j: (0, 0)),
            pl.BlockSpec((1, o1), lambda i, j: (0, 0)),
        ],
        out_specs=[
            pl.BlockSpec((1, 1, k * ts, o1), lambda i, j: (i, j, 0, 0)),
            pl.BlockSpec((2, o1), lambda i, j: (0, 0)),
        ],
        out_shape=[
            jax.ShapeDtypeStruct((b, nst, k * ts, o1), jnp.float32),
            jax.ShapeDtypeStruct((2, o1), jnp.float32),
        ],
    )(xyz, data_t, centers, w1t, b1)
    return y.reshape(b * s * k, o1), st


def _conv_body(x_ref, a_ref, c_ref, w_ref, b_ref, y_ref, st_ref):
    x = x_ref[...]
    z = jnp.maximum(x * a_ref[...] + c_ref[...], 0.0)
    y = jnp.dot(z, w_ref[...], preferred_element_type=jnp.float32) + b_ref[...]
    y_ref[...] = y

    @pl.when(pl.program_id(0) == 0)
    def _():
        st_ref[...] = jnp.zeros_like(st_ref)

    st_ref[0:1, :] += jnp.sum(y, axis=0, keepdims=True)
    st_ref[1:2, :] += jnp.sum(y * y, axis=0, keepdims=True)


def _conv_next(x, a, c, wt, bb, tr):
    r, cin = x.shape
    o = wt.shape[1]
    y, st = pl.pallas_call(
        _conv_body,
        grid=(r // tr,),
        in_specs=[
            pl.BlockSpec((tr, cin), lambda i: (i, 0)),
            pl.BlockSpec((1, cin), lambda i: (0, 0)),
            pl.BlockSpec((1, cin), lambda i: (0, 0)),
            pl.BlockSpec((cin, o), lambda i: (0, 0)),
            pl.BlockSpec((1, o), lambda i: (0, 0)),
        ],
        out_specs=[
            pl.BlockSpec((tr, o), lambda i: (i, 0)),
            pl.BlockSpec((2, o), lambda i: (0, 0)),
        ],
        out_shape=[
            jax.ShapeDtypeStruct((r, o), jnp.float32),
            jax.ShapeDtypeStruct((2, o), jnp.float32),
        ],
    )(x, a, c, wt, bb)
    return y, st


def _pool_body(k, ts, x_ref, a_ref, c_ref, o_ref):
    x = x_ref[0]
    z = jnp.maximum(x * a_ref[...] + c_ref[...], 0.0)
    m = z[0:ts, :]
    for j in range(1, k):
        m = jnp.maximum(m, z[j * ts:(j + 1) * ts, :])
    o_ref[0] = m


def _pool(y, a, c, b, s, k, ts):
    o = y.shape[-1]
    nst = (b * s) // ts
    y4 = y.reshape(nst, k * ts, o)
    out = pl.pallas_call(
        functools.partial(_pool_body, k, ts),
        grid=(nst,),
        in_specs=[
            pl.BlockSpec((1, k * ts, o), lambda i: (i, 0, 0)),
            pl.BlockSpec((1, o), lambda i: (0, 0)),
            pl.BlockSpec((1, o), lambda i: (0, 0)),
        ],
        out_specs=pl.BlockSpec((1, ts, o), lambda i: (i, 0, 0)),
        out_shape=jax.ShapeDtypeStruct((nst, ts, o), jnp.float32),
    )(y4, a, c)
    return out.reshape(b, s, o)


def _bn_affine(st, m, gamma, beta):
    mean = st[0] / m
    var = st[1] / m - mean * mean
    a = gamma / jnp.sqrt(var + 1e-5)
    c = beta - mean * a
    return a[None, :], c[None, :]


def _sa_layer(xyz, feats_t, centers, spec, branch_params, ts):
    b = xyz.shape[0]
    s = centers.shape[2]
    _, radius_list, nsample_list, _ = spec
    xyz_t = jnp.transpose(xyz, (0, 2, 1))
    if feats_t is None:
        data_t = jnp.concatenate([xyz_t, xyz_t], axis=2)
    else:
        data_t = jnp.concatenate([feats_t, xyz_t], axis=2)
    outs = []
    for i, radius in enumerate(radius_list):
        k = nsample_list[i]
        rsq = np.float32(radius ** 2)
        layers = branch_params[i]
        (w1, b1, g1, be1) = layers[0]
        y, st = _group_conv1(xyz, data_t, centers, jnp.transpose(w1),
                             b1[None, :], k, rsq, ts)
        m = np.float32(y.shape[0])
        a, c = _bn_affine(st, m, g1, be1)
        for (w, bb, gm, bt) in layers[1:]:
            y, st = _conv_next(y, a, c, jnp.transpose(w), bb[None, :],
                               min(2048, y.shape[0]))
            a, c = _bn_affine(st, m, gm, bt)
        pooled = _pool(y, a, c, b, s, k, ts)
        outs.append(jnp.transpose(pooled, (0, 2, 1)))
    return jnp.concatenate(outs, axis=1)


def kernel(xyz, params):
    l1_xyz = _fps(xyz, _SPECS[0][0])
    l1_pts = _sa_layer(xyz, None, l1_xyz, _SPECS[0], params[0], 128)

    l2_xyz = _fps(l1_xyz, _SPECS[1][0])
    l1_pts_t = jnp.transpose(l1_pts, (0, 2, 1))
    l2_pts = _sa_layer(l1_xyz, l1_pts_t, l2_xyz, _SPECS[1], params[1], 128)

    l3_xyz = _fps(l2_xyz, _SPECS[2][0])
    l2_pts_t = jnp.transpose(l2_pts, (0, 2, 1))
    l3_pts = _sa_layer(l2_xyz, l2_pts_t, l3_xyz, _SPECS[2], params[2], 128)

    return (l1_xyz, l2_xyz, l3_xyz, l3_pts, l2_pts, l1_pts)
